# baseline (speedup 1.0000x reference)
import functools

import jax
import jax.numpy as jnp
from jax import lax
from jax.experimental import pallas as pl
from jax.experimental.pallas import tpu as pltpu
from jax.experimental.pallas import tpu_sc as plsc

B = 16384
C = 1000

_SIGMA = 0.5
_SCALE = -1.0 / B

L = 16
NC = 2
NS = 16
NW = NC * NS
TW = 128
NSC_TILES = 24
SC_ITEMS = NSC_TILES * TW
ITEM0 = B - SC_ITEMS

BC = 1024
NBLK_TC = ITEM0 // BC
RCH = 8


def _tc_body(p_ref, t_ref, m_ref, out_ref):
    b = pl.program_id(0)

    @pl.when(b == 0)
    def _():
        out_ref[...] = jnp.zeros_like(out_ref)

    t = t_ref[0]
    m = m_ref[0]
    w = jnp.exp(-_SIGMA * m * m)
    w = jnp.where((m > 0.0) | (m < 0.0), w, 0.0)

    ri = lax.broadcasted_iota(jnp.int32, (RCH, BC), 0)
    acc = jnp.zeros((RCH, BC), jnp.float32)
    for k in range(C // RCH):
        pk = p_ref[pl.ds(k * RCH, RCH), :]
        acc = acc + jnp.where(ri == t - k * RCH, pk, 0.0)
    out_ref[...] += acc * w


_tc_reduce = pl.pallas_call(
    _tc_body,
    grid=(NBLK_TC,),
    in_specs=[
        pl.BlockSpec((C, BC), lambda b: (0, b)),
        pl.BlockSpec((1, 1, BC), lambda b: (b, 0, 0)),
        pl.BlockSpec((1, 1, BC), lambda b: (b, 0, 0)),
    ],
    out_specs=pl.BlockSpec((RCH, BC), lambda b: (0, 0)),
    out_shape=jax.ShapeDtypeStruct((RCH, BC), jnp.float32),
)


@functools.partial(
    pl.kernel,
    out_type=jax.ShapeDtypeStruct((NW, L), jnp.float32),
    mesh=plsc.VectorSubcoreMesh(core_axis_name="c", subcore_axis_name="s"),
    compiler_params=pltpu.CompilerParams(
        needs_layout_passes=False, skip_device_barrier=True),
    scratch_types=[
        pltpu.VMEM((C, TW), jnp.float32),
        pltpu.VMEM((TW,), jnp.int32),
        pltpu.VMEM((TW,), jnp.float32),
        pltpu.VMEM((L,), jnp.float32),
    ],
)
def _sc_gather(preds_t_hbm, targets_hbm, margin_hbm, out_hbm,
               buf_v, t_v, m_v, res_v):
    wid = lax.axis_index("s") * NC + lax.axis_index("c")
    res_v[...] = jnp.zeros((L,), jnp.float32)
    iota = lax.iota(jnp.int32, L)

    for rep in range((NSC_TILES + NW - 1) // NW):
        tile = wid + rep * NW

        @pl.when(tile < NSC_TILES)
        def _():
            c0 = pl.multiple_of(ITEM0 + tile * TW, TW)
            pltpu.sync_copy(preds_t_hbm.at[:, pl.ds(c0, TW)], buf_v)
            pltpu.sync_copy(targets_hbm.at[pl.ds(c0, TW)], t_v)
            pltpu.sync_copy(margin_hbm.at[pl.ds(c0, TW)], m_v)
            acc = jnp.zeros((L,), jnp.float32)
            for j in range(TW // L):
                t = t_v[pl.ds(j * L, L)]
                col = iota + j * L
                v = plsc.load_gather(buf_v, [t, col])
                m = m_v[pl.ds(j * L, L)]
                w = jnp.exp(-_SIGMA * m * m)
                nz = (m > 0.0) | (m < 0.0)
                acc = acc + jnp.where(nz, w * v, 0.0)
            res_v[...] = res_v[...] + acc

    pltpu.sync_copy(res_v, out_hbm.at[wid])


def kernel(preds, targets, margin):
    preds_t = preds.T
    targets = targets.astype(jnp.int32)
    t3 = targets.reshape(B // BC, 1, BC)
    m3 = margin.reshape(B // BC, 1, BC)
    tc_part = _tc_reduce(preds_t, t3, m3)
    sc_part = _sc_gather(preds_t, targets, margin)
    return (jnp.sum(tc_part) + jnp.sum(sc_part)) * _SCALE


# speedup vs baseline: 1.6681x; 1.6681x over previous
import jax
import jax.numpy as jnp
from jax import lax
from jax.experimental import pallas as pl
from jax.experimental.pallas import tpu as pltpu

B = 16384
C = 1000
BC = 4096
NBLK = B // BC
RCH = 8

_SIGMA = 0.5
_SCALE = -1.0 / B


def _tc_body(p_ref, t_ref, m_ref, out_ref):
    b = pl.program_id(0)

    @pl.when(b == 0)
    def _():
        out_ref[...] = jnp.zeros_like(out_ref)

    t = t_ref[0]
    m = m_ref[0]
    w = jnp.exp(-_SIGMA * m * m)
    w = jnp.where((m > 0.0) | (m < 0.0), w, 0.0)

    ri = lax.broadcasted_iota(jnp.int32, (RCH, BC), 0)
    acc = jnp.zeros((RCH, BC), jnp.float32)
    for k in range(C // RCH):
        pk = p_ref[pl.ds(k * RCH, RCH), :]
        acc = acc + jnp.where(ri == t - k * RCH, pk, 0.0)
    out_ref[...] += acc * w


_tc_reduce = pl.pallas_call(
    _tc_body,
    grid=(NBLK,),
    in_specs=[
        pl.BlockSpec((C, BC), lambda b: (0, b)),
        pl.BlockSpec((1, 1, BC), lambda b: (b, 0, 0)),
        pl.BlockSpec((1, 1, BC), lambda b: (b, 0, 0)),
    ],
    out_specs=pl.BlockSpec((RCH, BC), lambda b: (0, 0)),
    out_shape=jax.ShapeDtypeStruct((RCH, BC), jnp.float32),
)


def kernel(preds, targets, margin):
    preds_t = preds.T
    t3 = targets.astype(jnp.int32).reshape(NBLK, 1, BC)
    m3 = margin.reshape(NBLK, 1, BC)
    partials = _tc_reduce(preds_t, t3, m3)
    return jnp.sum(partials) * _SCALE
